# baseline (speedup 1.0000x reference)
import jax
import jax.numpy as jnp
from jax.experimental import pallas as pl
from jax.experimental.pallas import tpu as pltpu

_N_ENT = 1_000_000
_D = 32
_BLK = 41728
_NB = -(-_N_ENT // _BLK)
_TAIL = _N_ENT - (_NB - 1) * _BLK


def _huber(e):
    ae = jnp.abs(e)
    c = jnp.minimum(ae, 1.0)
    return c * (ae - 0.5 * c)


def _body(ent_ref, old_ent_ref, mask_hbm, tail_hbm, rel_ref, old_rel_ref,
          out_ref, acc_ref, mbuf_ref, sem_ref):
    i = pl.program_id(0)

    def _full_copy(step, slot):
        return pltpu.make_async_copy(
            mask_hbm.at[pl.ds(step * _BLK, _BLK)],
            mbuf_ref.at[slot],
            sem_ref.at[slot])

    def _tail_copy(slot):
        return pltpu.make_async_copy(
            tail_hbm.at[pl.ds(0, _BLK)],
            mbuf_ref.at[slot],
            sem_ref.at[slot])

    @pl.when(i == 0)
    def _init():
        _full_copy(0, 0).start()
        hr = _huber(rel_ref[...] - old_rel_ref[...])
        acc_ref[...] = jnp.zeros_like(acc_ref)
        acc_ref[0:1, 0:1] = jnp.sum(hr, keepdims=True).reshape(1, 1)

    @pl.when(i + 1 < _NB - 1)
    def _prefetch_full():
        _full_copy(i + 1, (i + 1) % 2).start()

    @pl.when(i + 1 == _NB - 1)
    def _prefetch_tail():
        _tail_copy((_NB - 1) % 2).start()

    @pl.when(i < _NB - 1)
    def _wait_full():
        _full_copy(i, i % 2).wait()

    @pl.when(i == _NB - 1)
    def _wait_tail():
        _tail_copy(i % 2).wait()

    m = mbuf_ref[i % 2].reshape(1, _BLK)
    m32 = jnp.broadcast_to(m, (_D, _BLK))
    h = _huber((ent_ref[...] - old_ent_ref[...]) * m32)

    @pl.when(i < _NB - 1)
    def _acc():
        acc_ref[...] += h

    @pl.when(i == _NB - 1)
    def _fin():
        total = acc_ref[...] + jnp.where(m32 > 0.0, h, 0.0)
        out_ref[...] = jnp.sum(total, keepdims=True).reshape(1, 1)


def kernel(ent_embeddings, rel_embeddings, old_ent_embeddings,
           old_rel_embeddings, entity_distill_mask):
    entT = ent_embeddings.T
    oldT = old_ent_embeddings.T
    relT = rel_embeddings.T
    old_relT = old_rel_embeddings.T
    tail = jnp.zeros((_BLK,), jnp.float32).at[:_TAIL].set(
        jax.lax.dynamic_slice(entity_distill_mask, ((_NB - 1) * _BLK,),
                              (_TAIL,)))
    out = pl.pallas_call(
        _body,
        grid=(_NB,),
        in_specs=[
            pl.BlockSpec((_D, _BLK), lambda i: (0, i)),
            pl.BlockSpec((_D, _BLK), lambda i: (0, i)),
            pl.BlockSpec(memory_space=pl.ANY),
            pl.BlockSpec(memory_space=pl.ANY),
            pl.BlockSpec(relT.shape, lambda i: (0, 0)),
            pl.BlockSpec(old_relT.shape, lambda i: (0, 0)),
        ],
        out_specs=pl.BlockSpec((1, 1), lambda i: (0, 0)),
        out_shape=jax.ShapeDtypeStruct((1, 1), jnp.float32),
        scratch_shapes=[
            pltpu.VMEM((_D, _BLK), jnp.float32),
            pltpu.VMEM((2, _BLK), jnp.float32),
            pltpu.SemaphoreType.DMA((2,)),
        ],
    )(entT, oldT, entity_distill_mask, tail, relT, old_relT)
    return out[0, 0]


# speedup vs baseline: 1.0201x; 1.0201x over previous
import jax
import jax.numpy as jnp
from jax.experimental import pallas as pl
from jax.experimental.pallas import tpu as pltpu

_N_ENT = 1_000_000
_D = 32
_BLK = 41728
_NB = -(-_N_ENT // _BLK)
_TAIL = _N_ENT - (_NB - 1) * _BLK


def _huber(e):
    ae = jnp.abs(e)
    c = jnp.minimum(ae, 1.0)
    return c * (ae - 0.5 * c)


def _body(ent_ref, old_ent_ref, mask_hbm, tail_hbm, rel_ref, old_rel_ref,
          out_ref, acc_ref, mbuf_ref, sem_ref):
    i = pl.program_id(0)

    def _full_copy(step, slot):
        return pltpu.make_async_copy(
            mask_hbm.at[pl.ds(step * _BLK, _BLK)],
            mbuf_ref.at[slot],
            sem_ref.at[slot])

    def _tail_copy(slot):
        return pltpu.make_async_copy(
            tail_hbm.at[pl.ds(0, _BLK)],
            mbuf_ref.at[slot],
            sem_ref.at[slot])

    @pl.when(i == 0)
    def _init():
        _full_copy(0, 0).start()
        hr = _huber(rel_ref[...] - old_rel_ref[...])
        acc_ref[...] = jnp.zeros_like(acc_ref)
        acc_ref[0:1, 0:1] = jnp.sum(hr, keepdims=True).reshape(1, 1)

    @pl.when(i + 1 < _NB - 1)
    def _prefetch_full():
        _full_copy(i + 1, (i + 1) % 2).start()

    @pl.when(i + 1 == _NB - 1)
    def _prefetch_tail():
        _tail_copy((_NB - 1) % 2).start()

    @pl.when(i < _NB - 1)
    def _wait_full():
        _full_copy(i, i % 2).wait()

    @pl.when(i == _NB - 1)
    def _wait_tail():
        _tail_copy(i % 2).wait()

    m = mbuf_ref[i % 2].reshape(1, _BLK)
    m8 = jnp.broadcast_to(m, (8, _BLK))
    d = (ent_ref[...] - old_ent_ref[...]).reshape(4, 8, _BLK)
    h = _huber(d * m8[None])

    @pl.when(i < _NB - 1)
    def _acc():
        acc_ref[...] += jnp.sum(h, axis=0)

    @pl.when(i == _NB - 1)
    def _fin():
        hg = jnp.sum(jnp.where(m8[None] > 0.0, h, 0.0), axis=0)
        out_ref[...] = jnp.sum(acc_ref[...] + hg, keepdims=True).reshape(1, 1)


def kernel(ent_embeddings, rel_embeddings, old_ent_embeddings,
           old_rel_embeddings, entity_distill_mask):
    entT = ent_embeddings.T
    oldT = old_ent_embeddings.T
    relT = rel_embeddings.T
    old_relT = old_rel_embeddings.T
    tail = jnp.zeros((_BLK,), jnp.float32).at[:_TAIL].set(
        jax.lax.dynamic_slice(entity_distill_mask, ((_NB - 1) * _BLK,),
                              (_TAIL,)))
    out = pl.pallas_call(
        _body,
        grid=(_NB,),
        in_specs=[
            pl.BlockSpec((_D, _BLK), lambda i: (0, i)),
            pl.BlockSpec((_D, _BLK), lambda i: (0, i)),
            pl.BlockSpec(memory_space=pl.ANY),
            pl.BlockSpec(memory_space=pl.ANY),
            pl.BlockSpec(relT.shape, lambda i: (0, 0)),
            pl.BlockSpec(old_relT.shape, lambda i: (0, 0)),
        ],
        out_specs=pl.BlockSpec((1, 1), lambda i: (0, 0)),
        out_shape=jax.ShapeDtypeStruct((1, 1), jnp.float32),
        scratch_shapes=[
            pltpu.VMEM((8, _BLK), jnp.float32),
            pltpu.VMEM((2, _BLK), jnp.float32),
            pltpu.SemaphoreType.DMA((2,)),
        ],
    )(entT, oldT, entity_distill_mask, tail, relT, old_relT)
    return out[0, 0]


# speedup vs baseline: 1.3254x; 1.2993x over previous
import jax
import jax.numpy as jnp
from jax.experimental import pallas as pl
from jax.experimental.pallas import tpu as pltpu

_N_ENT = 1_000_000
_D = 32
_BLK = 41728
_NB = -(-_N_ENT // _BLK)
_TAIL = _N_ENT - (_NB - 1) * _BLK


def _huber(e):
    ae = jnp.abs(e)
    c = jnp.minimum(ae, 1.0)
    return c * (ae - 0.5 * c)


def _body(ent_ref, old_ent_ref, mask_hbm, rel_ref, old_rel_ref,
          out_ref, acc_ref, mbuf_ref, sem_ref):
    i = pl.program_id(0)

    def _mask_copy(step, slot):
        return pltpu.make_async_copy(
            mask_hbm.at[pl.ds(step * _BLK, _BLK)],
            mbuf_ref.at[slot],
            sem_ref.at[slot])

    @pl.when(i == 0)
    def _init():
        _mask_copy(0, 0).start()
        hr = _huber(rel_ref[...] - old_rel_ref[...])
        acc_ref[...] = jnp.zeros_like(acc_ref)
        acc_ref[0:1, 0:1] = jnp.sum(hr, keepdims=True).reshape(1, 1)

    @pl.when(i + 1 < _NB)
    def _prefetch():
        _mask_copy(i + 1, (i + 1) % 2).start()

    _mask_copy(i, i % 2).wait()
    m = mbuf_ref[i % 2].reshape(1, _BLK)
    m8 = jnp.broadcast_to(m, (8, _BLK))
    d = (ent_ref[...] - old_ent_ref[...]).reshape(4, 8, _BLK)
    e = d * m8[None]
    h = jnp.where(m8[None] > 0.0, _huber(e), 0.0)
    acc_ref[...] += jnp.sum(h, axis=0)

    @pl.when(i == _NB - 1)
    def _fin():
        out_ref[...] = jnp.sum(acc_ref[...], keepdims=True).reshape(1, 1)


def kernel(ent_embeddings, rel_embeddings, old_ent_embeddings,
           old_rel_embeddings, entity_distill_mask):
    entT = ent_embeddings.T
    oldT = old_ent_embeddings.T
    relT = rel_embeddings.T
    old_relT = old_rel_embeddings.T
    mask_p = jnp.pad(entity_distill_mask, (0, _NB * _BLK - _N_ENT))
    out = pl.pallas_call(
        _body,
        grid=(_NB,),
        in_specs=[
            pl.BlockSpec((_D, _BLK), lambda i: (0, i)),
            pl.BlockSpec((_D, _BLK), lambda i: (0, i)),
            pl.BlockSpec(memory_space=pl.ANY),
            pl.BlockSpec(relT.shape, lambda i: (0, 0)),
            pl.BlockSpec(old_relT.shape, lambda i: (0, 0)),
        ],
        out_specs=pl.BlockSpec((1, 1), lambda i: (0, 0)),
        out_shape=jax.ShapeDtypeStruct((1, 1), jnp.float32),
        scratch_shapes=[
            pltpu.VMEM((8, _BLK), jnp.float32),
            pltpu.VMEM((2, _BLK), jnp.float32),
            pltpu.SemaphoreType.DMA((2,)),
        ],
    )(entT, oldT, mask_p, relT, old_relT)
    return out[0, 0]


# speedup vs baseline: 1.3678x; 1.0320x over previous
import jax
import jax.numpy as jnp
from jax.experimental import pallas as pl
from jax.experimental.pallas import tpu as pltpu

_N_ENT = 1_000_000
_D = 32
_BLK = 41728
_NB = -(-_N_ENT // _BLK)
_TAIL = _N_ENT - (_NB - 1) * _BLK


def _huber(e):
    ae = jnp.abs(e)
    c = jnp.minimum(ae, 1.0)
    return c * (ae - 0.5 * c)


def _body(ent_ref, old_ent_ref, mask_hbm, tail_hbm, rel_ref, old_rel_ref,
          out_ref, acc_ref, mbuf_ref, sem_ref):
    i = pl.program_id(0)

    def _mask_copy(step, slot):
        return pltpu.make_async_copy(
            mask_hbm.at[pl.ds(step * _BLK, _BLK)],
            mbuf_ref.at[slot],
            sem_ref.at[slot])

    def _tail_copy(slot):
        return pltpu.make_async_copy(
            tail_hbm.at[pl.ds(0, _BLK)],
            mbuf_ref.at[slot],
            sem_ref.at[slot])

    @pl.when(i == 0)
    def _init():
        _mask_copy(0, 0).start()
        hr = _huber(rel_ref[...] - old_rel_ref[...])
        acc_ref[...] = jnp.zeros_like(acc_ref)
        acc_ref[0:1, 0:1] = jnp.sum(hr, keepdims=True).reshape(1, 1)

    @pl.when(i + 1 < _NB - 1)
    def _prefetch():
        _mask_copy(i + 1, (i + 1) % 2).start()

    @pl.when(i + 1 == _NB - 1)
    def _prefetch_tail():
        _tail_copy((_NB - 1) % 2).start()

    _mask_copy(0, i % 2).wait()
    m = mbuf_ref[i % 2].reshape(1, _BLK)
    m8 = jnp.broadcast_to(m, (8, _BLK))
    d = (ent_ref[...] - old_ent_ref[...]).reshape(4, 8, _BLK)
    e = d * m8[None]
    h = jnp.where(m8[None] > 0.0, _huber(e), 0.0)
    acc_ref[...] += jnp.sum(h, axis=0)

    @pl.when(i == _NB - 1)
    def _fin():
        out_ref[...] = jnp.sum(acc_ref[...], keepdims=True).reshape(1, 1)


def kernel(ent_embeddings, rel_embeddings, old_ent_embeddings,
           old_rel_embeddings, entity_distill_mask):
    entT = ent_embeddings.T
    oldT = old_ent_embeddings.T
    relT = rel_embeddings.T
    old_relT = old_rel_embeddings.T
    tail = jnp.zeros((_BLK,), jnp.float32).at[:_TAIL].set(
        entity_distill_mask[(_NB - 1) * _BLK:])
    out = pl.pallas_call(
        _body,
        grid=(_NB,),
        in_specs=[
            pl.BlockSpec((_D, _BLK), lambda i: (0, i)),
            pl.BlockSpec((_D, _BLK), lambda i: (0, i)),
            pl.BlockSpec(memory_space=pl.ANY),
            pl.BlockSpec(memory_space=pl.ANY),
            pl.BlockSpec(relT.shape, lambda i: (0, 0)),
            pl.BlockSpec(old_relT.shape, lambda i: (0, 0)),
        ],
        out_specs=pl.BlockSpec((1, 1), lambda i: (0, 0)),
        out_shape=jax.ShapeDtypeStruct((1, 1), jnp.float32),
        scratch_shapes=[
            pltpu.VMEM((8, _BLK), jnp.float32),
            pltpu.VMEM((2, _BLK), jnp.float32),
            pltpu.SemaphoreType.DMA((2,)),
        ],
    )(entT, oldT, entity_distill_mask, tail, relT, old_relT)
    return out[0, 0]
